# Initial kernel scaffold; baseline (speedup 1.0000x reference)
import jax
import jax.numpy as jnp
from jax.experimental import pallas as pl


def kernel(x, step, pe):
    raise NotImplementedError("write your pallas kernel here")


# speedup vs baseline: 1.8062x; 1.8062x over previous
import math

import jax
import jax.numpy as jnp
from jax import lax
from jax.experimental import pallas as pl
from jax.experimental.pallas import tpu as pltpu
from jax.experimental.pallas import tpu_sc as plsc

D = 128
L = 16
SCALE = math.sqrt(float(D))


def _make_sc_kernel(B, NC, NS):
    NW = NC * NS
    b_per_w = B // NW
    CH = 128
    NCH = b_per_w // CH
    mesh = plsc.VectorSubcoreMesh(core_axis_name="c", subcore_axis_name="s")

    def body(x_hbm, step_hbm, pe_hbm, out_hbm, idx_v, x_v, pe_v, gsem):
        wid = lax.axis_index("s") * NC + lax.axis_index("c")
        pltpu.sync_copy(step_hbm.at[wid], idx_v)

        def do_chunk(c):
            base = wid * b_per_w + c * CH
            gather = pltpu.async_copy(pe_hbm.at[idx_v.at[c]], pe_v, gsem)
            pltpu.sync_copy(x_hbm.at[pl.ds(base, CH), :], x_v)
            gather.wait()

            def row(r, carry):
                for g in range(D // L):
                    sl = pl.ds(g * L, L)
                    x_v[r, sl] = x_v[r, sl] * SCALE + pe_v[r, sl]
                return carry

            lax.fori_loop(0, CH, row, 0)
            pltpu.sync_copy(x_v, out_hbm.at[pl.ds(base, CH), :])

        for c in range(NCH):
            do_chunk(c)

    return pl.kernel(
        body,
        out_type=jax.ShapeDtypeStruct((B, D), jnp.float32),
        mesh=mesh,
        scratch_types=[
            pltpu.VMEM((NCH, CH), jnp.int32),
            pltpu.VMEM((CH, D), jnp.float32),
            pltpu.VMEM((CH, D), jnp.float32),
            pltpu.SemaphoreType.DMA,
        ],
    )


def kernel(x, step, pe):
    B = x.shape[0]
    info = plsc.get_sparse_core_info()
    NC, NS = info.num_cores, info.num_subcores
    NW = NC * NS
    b_per_w = B // NW
    CH = 128
    step3 = step.astype(jnp.int32).reshape(NW, b_per_w // CH, CH)
    return _make_sc_kernel(B, NC, NS)(x, step3, pe)


# speedup vs baseline: 1.9700x; 1.0907x over previous
import math

import jax
import jax.numpy as jnp
from jax import lax
from jax.experimental import pallas as pl
from jax.experimental.pallas import tpu as pltpu
from jax.experimental.pallas import tpu_sc as plsc

D = 128
L = 16
SCALE = math.sqrt(float(D))


def _make_sc_kernel(B, NC, NS):
    NW = NC * NS
    b_per_w = B // NW
    CH = 128
    NCH = b_per_w // CH
    mesh = plsc.VectorSubcoreMesh(core_axis_name="c", subcore_axis_name="s")

    def body(x_hbm, step_hbm, pe_hbm, out_hbm, idx_v,
             xv0, xv1, pv0, pv1, ov0, ov1, ls0, ls1, ws0, ws1):
        wid = lax.axis_index("s") * NC + lax.axis_index("c")
        base = wid * b_per_w
        pltpu.sync_copy(step_hbm.at[wid], idx_v)
        xv, pv, ov = (xv0, xv1), (pv0, pv1), (ov0, ov1)
        ls, ws = (ls0, ls1), (ws0, ws1)

        def issue_loads(c):
            b = c & 1
            g = pltpu.async_copy(pe_hbm.at[idx_v.at[c]], pv[b], ls[b])
            xc = pltpu.async_copy(x_hbm.at[pl.ds(base + c * CH, CH), :],
                                  xv[b], ls[b])
            return (g, xc)

        pending = issue_loads(0)
        wb = [None, None]
        for c in range(NCH):
            b = c & 1
            nxt = issue_loads(c + 1) if c + 1 < NCH else None
            pending[0].wait()
            pending[1].wait()
            if wb[b] is not None:
                wb[b].wait()

            def row(r, carry, b=b):
                for g in range(D // L):
                    sl = pl.ds(g * L, L)
                    ov[b][r, sl] = xv[b][r, sl] * SCALE + pv[b][r, sl]
                return carry

            lax.fori_loop(0, CH, row, 0)
            wb[b] = pltpu.async_copy(ov[b],
                                     out_hbm.at[pl.ds(base + c * CH, CH), :],
                                     ws[b])
            pending = nxt
        wb[0].wait()
        wb[1].wait()

    buf = pltpu.VMEM((CH, D), jnp.float32)
    return pl.kernel(
        body,
        out_type=jax.ShapeDtypeStruct((B, D), jnp.float32),
        mesh=mesh,
        scratch_types=[
            pltpu.VMEM((NCH, CH), jnp.int32),
            buf, buf, buf, buf, buf, buf,
            pltpu.SemaphoreType.DMA,
            pltpu.SemaphoreType.DMA,
            pltpu.SemaphoreType.DMA,
            pltpu.SemaphoreType.DMA,
        ],
    )


def kernel(x, step, pe):
    B = x.shape[0]
    info = plsc.get_sparse_core_info()
    NC, NS = info.num_cores, info.num_subcores
    NW = NC * NS
    b_per_w = B // NW
    CH = 128
    step3 = step.astype(jnp.int32).reshape(NW, b_per_w // CH, CH)
    return _make_sc_kernel(B, NC, NS)(x, step3, pe)


# speedup vs baseline: 1.9999x; 1.0152x over previous
import math

import jax
import jax.numpy as jnp
from jax import lax
from jax.experimental import pallas as pl
from jax.experimental.pallas import tpu as pltpu
from jax.experimental.pallas import tpu_sc as plsc

D = 128
L = 16
SCALE = math.sqrt(float(D))


def _make_sc_kernel(B, NC, NS):
    NW = NC * NS
    b_per_w = B // NW
    CH = 128
    NCH = b_per_w // CH
    mesh = plsc.VectorSubcoreMesh(core_axis_name="c", subcore_axis_name="s")

    V_PAD = 1024
    rows_per_tile = V_PAD // NS

    def body(x_hbm, step_hbm, pe_hbm, out_hbm, idx_v,
             xv0, xv1, pv0, pv1, ov0, ov1, ls0, ls1, ws0, ws1):
        wid = lax.axis_index("s") * NC + lax.axis_index("c")
        sid = lax.axis_index("s")
        base = wid * b_per_w
        xv, pv, ov = (xv0, xv1), (pv0, pv1), (ov0, ov1)
        ls, ws = (ls0, ls1), (ws0, ws1)

        x0 = pltpu.async_copy(x_hbm.at[pl.ds(base, CH), :], xv[0], ls[0])
        x1 = pltpu.async_copy(x_hbm.at[pl.ds(base + CH, CH), :], xv[1], ls[1])
        pltpu.sync_copy(step_hbm.at[wid], idx_v)

        def issue(c):
            b = c & 1
            g = pltpu.async_copy(pe_hbm.at[idx_v.at[c]], pv[b], ls[b])
            xc = (pltpu.async_copy(x_hbm.at[pl.ds(base + c * CH, CH), :],
                                   xv[b], ls[b])
                  if c >= 2 else (x0 if c == 0 else x1))
            return (g, xc)

        pending = issue(0)
        wb = [None, None]
        for c in range(NCH):
            b = c & 1
            nxt = issue(c + 1) if c + 1 < NCH else None
            pending[0].wait()
            pending[1].wait()
            if wb[b] is not None:
                wb[b].wait()

            def row(r, carry, b=b):
                for g in range(D // L):
                    sl = pl.ds(g * L, L)
                    ov[b][r, sl] = xv[b][r, sl] * SCALE + pv[b][r, sl]
                return carry

            lax.fori_loop(0, CH, row, 0)
            wb[b] = pltpu.async_copy(ov[b],
                                     out_hbm.at[pl.ds(base + c * CH, CH), :],
                                     ws[b])
            pending = nxt
        wb[0].wait()
        wb[1].wait()

    buf = pltpu.VMEM((CH, D), jnp.float32)
    return pl.kernel(
        body,
        out_type=jax.ShapeDtypeStruct((B, D), jnp.float32),
        mesh=mesh,
        scratch_types=[
            pltpu.VMEM((NCH, CH), jnp.int32),
            buf, buf, buf, buf, buf, buf,
            pltpu.SemaphoreType.DMA,
            pltpu.SemaphoreType.DMA,
            pltpu.SemaphoreType.DMA,
            pltpu.SemaphoreType.DMA,
        ],
    )


def kernel(x, step, pe):
    B = x.shape[0]
    info = plsc.get_sparse_core_info()
    NC, NS = info.num_cores, info.num_subcores
    NW = NC * NS
    b_per_w = B // NW
    CH = 128
    step3 = step.astype(jnp.int32).reshape(NW, b_per_w // CH, CH)
    pe_pad = jnp.pad(pe, ((0, 1024 - pe.shape[0]), (0, 0)))
    return _make_sc_kernel(B, NC, NS)(x, step3, pe_pad)


# speedup vs baseline: 2.0079x; 1.0040x over previous
import math

import jax
import jax.numpy as jnp
from jax import lax
from jax.experimental import pallas as pl
from jax.experimental.pallas import tpu as pltpu
from jax.experimental.pallas import tpu_sc as plsc

D = 128
L = 16
SCALE = math.sqrt(float(D))


def _make_sc_kernel(B, NC, NS):
    NW = NC * NS
    b_per_w = B // NW
    CH = 128
    NCH = b_per_w // CH
    mesh = plsc.VectorSubcoreMesh(core_axis_name="c", subcore_axis_name="s")

    V_PAD = 1024
    rows_per_tile = V_PAD // NS

    def body(x_hbm, step_hbm, pe_hbm, out_hbm, idx_v,
             xv0, xv1, pv0, pv1, ov0, ov1, ls0, ls1, ws0, ws1):
        wid = lax.axis_index("s") * NC + lax.axis_index("c")
        sid = lax.axis_index("s")
        base = wid * b_per_w
        xv, pv, ov = (xv0, xv1), (pv0, pv1), (ov0, ov1)
        ls, ws = (ls0, ls1), (ws0, ws1)

        x0 = pltpu.async_copy(x_hbm.at[pl.ds(base, CH), :], xv[0], ls[0])
        x1 = pltpu.async_copy(x_hbm.at[pl.ds(base + CH, CH), :], xv[1], ls[1])
        pltpu.sync_copy(step_hbm.at[wid], idx_v)

        def issue(c):
            b = c & 1
            g = pltpu.async_copy(pe_hbm.at[idx_v.at[c]], pv[b], ls[b])
            xc = (pltpu.async_copy(x_hbm.at[pl.ds(base + c * CH, CH), :],
                                   xv[b], ls[b])
                  if c >= 2 else (x0 if c == 0 else x1))
            return (g, xc)

        pending = issue(0)
        wb = [None, None]
        for c in range(NCH):
            b = c & 1
            nxt = issue(c + 1) if c + 1 < NCH else None
            pending[0].wait()
            pending[1].wait()
            if wb[b] is not None:
                wb[b].wait()

            def row(r, carry, b=b):
                for g in range(D // L):
                    sl = pl.ds(g * L, L)
                    ov[b][r, sl] = xv[b][r, sl] * SCALE + pv[b][r, sl]
                return carry

            lax.fori_loop(0, CH, row, 0)
            wb[b] = pltpu.async_copy(ov[b],
                                     out_hbm.at[pl.ds(base + c * CH, CH), :],
                                     ws[b])
            pending = nxt
        wb[0].wait()
        wb[1].wait()

    buf = pltpu.VMEM((CH, D), jnp.float32)
    return pl.kernel(
        body,
        out_type=jax.ShapeDtypeStruct((B, D), jnp.float32),
        mesh=mesh,
        scratch_types=[
            pltpu.VMEM((NCH, CH), jnp.int32),
            buf, buf, buf, buf, buf, buf,
            pltpu.SemaphoreType.DMA,
            pltpu.SemaphoreType.DMA,
            pltpu.SemaphoreType.DMA,
            pltpu.SemaphoreType.DMA,
        ],
    )


def kernel(x, step, pe):
    B = x.shape[0]
    info = plsc.get_sparse_core_info()
    NC, NS = info.num_cores, info.num_subcores
    NW = NC * NS
    b_per_w = B // NW
    CH = 128
    step3 = step.astype(jnp.int32).reshape(NW, b_per_w // CH, CH)
    return _make_sc_kernel(B, NC, NS)(x, step3, pe)
